# Initial kernel scaffold; baseline (speedup 1.0000x reference)
import jax
import jax.numpy as jnp
from jax.experimental import pallas as pl


def kernel(x, edge_index, batch, W_gcn, b_gcn, W_gat, att_src, att_dst, b_gat, W_sage_l, b_sage_l, W_sage_r, W_gin, b_gin, W_lin, b_lin, W_com, b_com, W_glob, b_glob):
    raise NotImplementedError("write your pallas kernel here")


# speedup vs baseline: 16.3943x; 16.3943x over previous
import functools
import jax
import jax.numpy as jnp
from jax import lax
from jax.experimental import pallas as pl
from jax.experimental.pallas import tpu as pltpu
from jax.experimental.pallas import tpu_sc as plsc

N = 10000
NP = 10240
E = 320000
D = 128
COM = 100
BN = 512
GRID = NP // BN
RPT = NP // 16

K = 80
EPT1 = E // 16
CH1 = EPT1 // K
EPT2 = E // 32
CH2 = EPT2 // K
KH = 400
EH = E // 32
CHH = EH // KH

f32 = jnp.float32
i32 = jnp.int32


def _leaky(v):
    return jnp.where(v >= 0, v, 0.2 * v)


def _tc1_body(x_ref, b_ref, wgat_ref, asrc_ref, adst_ref,
              as_ref, ad_ref, amax_ref, csum_ref, ccnt_ref, gsum_ref):
    i = pl.program_id(0)
    xb = x_ref[...]
    xg = jnp.dot(xb, wgat_ref[...], preferred_element_type=f32)
    a_s = lax.dot_general(asrc_ref[...], xg, (((1,), (1,)), ((), ())),
                          preferred_element_type=f32)
    a_d = lax.dot_general(adst_ref[...], xg, (((1,), (1,)), ((), ())),
                          preferred_element_type=f32)
    as_ref[...] = a_s
    ad_ref[...] = a_d
    bcol = b_ref[...]
    oh = (bcol == lax.broadcasted_iota(i32, (BN, COM), 1)).astype(f32)
    csum_b = lax.dot_general(oh, xb, (((0,), (0,)), ((), ())),
                             preferred_element_type=f32)
    ccnt_b = lax.dot_general(oh, jnp.ones((BN, 1), f32),
                             (((0,), (0,)), ((), ())),
                             preferred_element_type=f32)
    gsum_b = jnp.sum(xb, axis=0, keepdims=True)
    a_b = jnp.maximum(jnp.max(a_s), 0.0).reshape(1, 1)

    @pl.when(i == 0)
    def _():
        csum_ref[...] = csum_b
        ccnt_ref[...] = ccnt_b
        gsum_ref[...] = gsum_b
        amax_ref[...] = a_b

    @pl.when(i > 0)
    def _():
        csum_ref[...] += csum_b
        ccnt_ref[...] += ccnt_b
        gsum_ref[...] += gsum_b
        amax_ref[...] = jnp.maximum(amax_ref[...], a_b)


_tc1 = pl.pallas_call(
    _tc1_body,
    grid=(GRID,),
    in_specs=[
        pl.BlockSpec((BN, D), lambda i: (i, 0)),
        pl.BlockSpec((BN, 1), lambda i: (i, 0)),
        pl.BlockSpec((D, D), lambda i: (0, 0)),
        pl.BlockSpec((1, D), lambda i: (0, 0)),
        pl.BlockSpec((1, D), lambda i: (0, 0)),
    ],
    out_specs=[
        pl.BlockSpec((1, BN), lambda i: (0, i)),
        pl.BlockSpec((1, BN), lambda i: (0, i)),
        pl.BlockSpec((1, 1), lambda i: (0, 0)),
        pl.BlockSpec((COM, D), lambda i: (0, 0)),
        pl.BlockSpec((COM, 1), lambda i: (0, 0)),
        pl.BlockSpec((1, D), lambda i: (0, 0)),
    ],
    out_shape=[
        jax.ShapeDtypeStruct((1, NP), f32),
        jax.ShapeDtypeStruct((1, NP), f32),
        jax.ShapeDtypeStruct((1, 1), f32),
        jax.ShapeDtypeStruct((COM, D), f32),
        jax.ShapeDtypeStruct((COM, 1), f32),
        jax.ShapeDtypeStruct((1, D), f32),
    ],
    compiler_params=pltpu.CompilerParams(
        dimension_semantics=("arbitrary",)),
)


def _tc2_body(x_ref, cnt0_ref, cnt1_ref, ad_ref, amax_ref,
              csum_ref, ccnt_ref, gsum_ref, wcom_ref, bcom_ref, wglob_ref,
              bglob_ref, x2_ref, c_ref, com_ref, gvec_ref):
    i = pl.program_id(0)
    cnt = cnt0_ref[...] + cnt1_ref[...]
    dinv = lax.rsqrt(cnt + 1.0)
    x2_ref[...] = x_ref[...] * dinv
    c_ref[...] = _leaky(amax_ref[0, 0] + ad_ref[...])

    @pl.when(i == 0)
    def _():
        cmean = csum_ref[...] / jnp.maximum(ccnt_ref[...], 1.0)
        com_ref[...] = jnp.dot(cmean, wcom_ref[...],
                               preferred_element_type=f32) + bcom_ref[...]
        gvec_ref[...] = jnp.dot(gsum_ref[...] * (1.0 / N), wglob_ref[...],
                                preferred_element_type=f32) + bglob_ref[...]


_tc2 = pl.pallas_call(
    _tc2_body,
    grid=(GRID,),
    in_specs=[
        pl.BlockSpec((BN, D), lambda i: (i, 0)),
        pl.BlockSpec((BN, 1), lambda i: (i, 0)),
        pl.BlockSpec((BN, 1), lambda i: (i, 0)),
        pl.BlockSpec((1, BN), lambda i: (0, i)),
        pl.BlockSpec((1, 1), lambda i: (0, 0)),
        pl.BlockSpec((COM, D), lambda i: (0, 0)),
        pl.BlockSpec((COM, 1), lambda i: (0, 0)),
        pl.BlockSpec((1, D), lambda i: (0, 0)),
        pl.BlockSpec((D, D), lambda i: (0, 0)),
        pl.BlockSpec((1, D), lambda i: (0, 0)),
        pl.BlockSpec((D, D), lambda i: (0, 0)),
        pl.BlockSpec((1, D), lambda i: (0, 0)),
    ],
    out_specs=[
        pl.BlockSpec((BN, D), lambda i: (i, 0)),
        pl.BlockSpec((1, BN), lambda i: (0, i)),
        pl.BlockSpec((COM, D), lambda i: (0, 0)),
        pl.BlockSpec((1, D), lambda i: (0, 0)),
    ],
    out_shape=[
        jax.ShapeDtypeStruct((NP, D), f32),
        jax.ShapeDtypeStruct((1, NP), f32),
        jax.ShapeDtypeStruct((COM, D), f32),
        jax.ShapeDtypeStruct((1, D), f32),
    ],
    compiler_params=pltpu.CompilerParams(
        dimension_semantics=("arbitrary",)),
)


def _sc_hist_body(dst_hbm, cnt0_hbm, cnt1_hbm, idx_v, ones_v, zb_v, acc_sh):
    cid = lax.axis_index("c")
    sid = lax.axis_index("s")
    wid = sid * 2 + cid
    for g in range(KH // 16):
        ones_v[pl.ds(g * 16, 16)] = jnp.ones((16,), f32)
    for g in range(RPT // 16):
        zb_v[pl.ds(g * 16, 16)] = jnp.zeros((16,), f32)
    pltpu.sync_copy(zb_v, acc_sh.at[pl.ds(sid * RPT, RPT)])
    plsc.subcore_barrier()

    def chunk(j, carry):
        base = wid * EH + j * KH
        pltpu.sync_copy(dst_hbm.at[pl.ds(base, KH)], idx_v)
        pltpu.sync_copy(ones_v, acc_sh.at[idx_v], add=True)
        return carry

    lax.fori_loop(0, CHH, chunk, 0)
    plsc.subcore_barrier()
    sl = pl.ds(sid * RPT, RPT)

    @pl.when(cid == 0)
    def _():
        pltpu.sync_copy(acc_sh.at[sl], cnt0_hbm.at[sl])

    @pl.when(cid == 1)
    def _():
        pltpu.sync_copy(acc_sh.at[sl], cnt1_hbm.at[sl])


@functools.lru_cache(maxsize=None)
def _get_sc_hist():
    return pl.kernel(
        _sc_hist_body,
        mesh=plsc.VectorSubcoreMesh(core_axis_name="c",
                                    subcore_axis_name="s"),
        out_type=[
            jax.ShapeDtypeStruct((NP,), f32),
            jax.ShapeDtypeStruct((NP,), f32),
        ],
        scratch_types=[
            pltpu.VMEM((KH,), i32),
            pltpu.VMEM((KH,), f32),
            pltpu.VMEM((RPT,), f32),
            pltpu.VMEM_SHARED((NP,), f32),
        ],
        compiler_params=pltpu.CompilerParams(needs_layout_passes=False,
                                             use_tc_tiling_on_sc=False),
    )


def _sc_main_body(x_hbm, x2_hbm, src_hbm, dst_hbm, as_hbm, ad_hbm, c_hbm,
                  aggx_hbm, gatw_hbm, x2a0_hbm, x2a1_hbm, den_hbm,
                  srcv, dstv, rows, wbuf, asv, adv, cv, gsem, ssem,
                  acc_sh, den_sh):
    cid = lax.axis_index("c")
    sid = lax.axis_index("s")
    sl = pl.ds(sid * RPT, RPT)

    def zero_acc():
        def zrow(r, carry):
            for g in range(D // 16):
                rows[r, pl.ds(g * 16, 16)] = jnp.zeros((16,), f32)
            return carry

        lax.fori_loop(0, K, zrow, 0)
        for j in range(RPT // K):
            pltpu.sync_copy(rows, acc_sh.at[pl.ds(sid * RPT + j * K, K)])

    zero_acc()
    for g in range(K // 16):
        wbuf[pl.ds(g * 16, 16)] = jnp.zeros((16,), f32)
    for j in range(RPT // K):
        pltpu.sync_copy(wbuf, den_sh.at[pl.ds(sid * RPT + j * K, K)])
    plsc.subcore_barrier()

    def load_idx(base):
        pltpu.sync_copy(src_hbm.at[pl.ds(base, K)], srcv)
        pltpu.sync_copy(dst_hbm.at[pl.ds(base, K)], dstv)

    @pl.when(cid == 0)
    def _():
        def chunk(ci, carry):
            load_idx(sid * EPT1 + ci * K)
            pltpu.async_copy(x_hbm.at[srcv], rows, gsem).wait()
            pltpu.sync_copy(rows, acc_sh.at[dstv], add=True)
            return carry

        lax.fori_loop(0, CH1, chunk, 0)

    @pl.when(cid == 1)
    def _():
        def chunk(ci, carry):
            load_idx(sid * EPT1 + ci * K)
            cp_r = pltpu.async_copy(x_hbm.at[srcv], rows, gsem)
            cp_a = pltpu.async_copy(as_hbm.at[srcv], asv, ssem)
            cp_b = pltpu.async_copy(ad_hbm.at[dstv], adv, ssem)
            cp_c = pltpu.async_copy(c_hbm.at[dstv], cv, ssem)
            cp_a.wait()
            cp_b.wait()
            cp_c.wait()
            cp_r.wait()
            for g in range(K // 16):
                o = g * 16
                ev = asv[pl.ds(o, 16)] + adv[pl.ds(o, 16)]
                ev = jnp.where(ev >= 0, ev, 0.2 * ev)
                w16 = jnp.exp(ev - cv[pl.ds(o, 16)])
                wbuf[pl.ds(o, 16)] = w16
                for j in range(16):
                    ws = jnp.full((16,), w16[j], f32)
                    e = o + j
                    for cg in range(D // 16):
                        rows[e, pl.ds(cg * 16, 16)] = \
                            rows[e, pl.ds(cg * 16, 16)] * ws
            pltpu.sync_copy(wbuf, den_sh.at[dstv], add=True)
            pltpu.sync_copy(rows, acc_sh.at[dstv], add=True)
            return carry

        lax.fori_loop(0, CH1, chunk, 0)

    plsc.subcore_barrier()

    @pl.when(cid == 0)
    def _():
        pltpu.sync_copy(acc_sh.at[sl], aggx_hbm.at[sl])

    @pl.when(cid == 1)
    def _():
        pltpu.sync_copy(acc_sh.at[sl], gatw_hbm.at[sl])
        pltpu.sync_copy(den_sh.at[sl], den_hbm.at[sl])

    zero_acc()
    plsc.subcore_barrier()

    def chunk2(ci, carry):
        load_idx(cid * (E // 2) + sid * EPT2 + ci * K)
        pltpu.async_copy(x2_hbm.at[srcv], rows, gsem).wait()
        pltpu.sync_copy(rows, acc_sh.at[dstv], add=True)
        return carry

    lax.fori_loop(0, CH2, chunk2, 0)
    plsc.subcore_barrier()

    @pl.when(cid == 0)
    def _():
        pltpu.sync_copy(acc_sh.at[sl], x2a0_hbm.at[sl])

    @pl.when(cid == 1)
    def _():
        pltpu.sync_copy(acc_sh.at[sl], x2a1_hbm.at[sl])


@functools.lru_cache(maxsize=None)
def _get_sc_main():
    return pl.kernel(
        _sc_main_body,
        mesh=plsc.VectorSubcoreMesh(core_axis_name="c",
                                    subcore_axis_name="s"),
        out_type=[
            jax.ShapeDtypeStruct((NP, D), f32),
            jax.ShapeDtypeStruct((NP, D), f32),
            jax.ShapeDtypeStruct((NP, D), f32),
            jax.ShapeDtypeStruct((NP, D), f32),
            jax.ShapeDtypeStruct((NP,), f32),
        ],
        scratch_types=[
            pltpu.VMEM((K,), i32),
            pltpu.VMEM((K,), i32),
            pltpu.VMEM((K, D), f32),
            pltpu.VMEM((K,), f32),
            pltpu.VMEM((K,), f32),
            pltpu.VMEM((K,), f32),
            pltpu.VMEM((K,), f32),
            pltpu.SemaphoreType.DMA,
            pltpu.SemaphoreType.DMA,
            pltpu.VMEM_SHARED((NP, D), f32),
            pltpu.VMEM_SHARED((NP,), f32),
        ],
        compiler_params=pltpu.CompilerParams(needs_layout_passes=False,
                                             use_tc_tiling_on_sc=False),
    )


def _tc3_body(x_ref, aggx_ref, gatw_ref, x2a0_ref, x2a1_ref, den_ref,
              cnt0_ref, cnt1_ref, as_ref, ad_ref, c_ref, b_ref,
              com_ref, gvec_ref, wgcn_ref, wgat_ref,
              wsl_ref, wsr_ref, wgin_ref, wlin_ref,
              bgcn_ref, bgat_ref, bsl_ref, bgin_ref, blin_ref, out_ref):
    xb = x_ref[...]
    aggx = aggx_ref[...]
    cnt = cnt0_ref[...] + cnt1_ref[...]
    dinv = lax.rsqrt(cnt + 1.0)
    x2b = xb * dinv
    gcn = dinv * jnp.dot(x2a0_ref[...] + x2a1_ref[...] + x2b, wgcn_ref[...],
                         preferred_element_type=f32) + bgcn_ref[...]
    es = jnp.exp(_leaky(as_ref[...] + ad_ref[...]) - c_ref[...])
    gat = jnp.dot(gatw_ref[...] + es * xb, wgat_ref[...],
                  preferred_element_type=f32) / (den_ref[...] + es) \
        + bgat_ref[...]
    invc = 1.0 / jnp.maximum(cnt, 1.0)
    sage = jnp.dot(aggx * invc, wsl_ref[...], preferred_element_type=f32) \
        + bsl_ref[...] \
        + jnp.dot(xb, wsr_ref[...], preferred_element_type=f32)
    gin = jnp.dot(xb + aggx, wgin_ref[...],
                  preferred_element_type=f32) + bgin_ref[...]
    lin = jnp.dot(xb, wlin_ref[...],
                  preferred_element_type=f32) + blin_ref[...]
    oh = (b_ref[...] == lax.broadcasted_iota(i32, (BN, COM), 1)).astype(f32)
    comm = jnp.dot(oh, com_ref[...], preferred_element_type=f32)
    glob = jnp.broadcast_to(gvec_ref[...], (BN, D))
    r = lambda v: jnp.maximum(v, 0.0)
    out_ref[...] = 0.12 * (r(gcn) + r(gat) + r(sage) + r(gin) + r(lin)) \
        + 0.3 * r(comm) + 0.1 * r(glob)


_col = pl.BlockSpec((BN, 1), lambda i: (i, 0))
_mat = pl.BlockSpec((BN, D), lambda i: (i, 0))
_wmat = pl.BlockSpec((D, D), lambda i: (0, 0))
_brow = pl.BlockSpec((1, D), lambda i: (0, 0))

_tc3 = pl.pallas_call(
    _tc3_body,
    grid=(GRID,),
    in_specs=[
        _mat, _mat, _mat, _mat, _mat,
        _col, _col, _col, _col, _col, _col,
        pl.BlockSpec((BN, 1), lambda i: (i, 0)),
        pl.BlockSpec((COM, D), lambda i: (0, 0)),
        _brow,
        _wmat, _wmat, _wmat, _wmat, _wmat, _wmat,
        _brow, _brow, _brow, _brow, _brow,
    ],
    out_specs=[_mat],
    out_shape=[jax.ShapeDtypeStruct((NP, D), f32)],
    compiler_params=pltpu.CompilerParams(
        dimension_semantics=("arbitrary",)),
)


def kernel(x, edge_index, batch, W_gcn, b_gcn, W_gat, att_src, att_dst,
           b_gat, W_sage_l, b_sage_l, W_sage_r, W_gin, b_gin, W_lin, b_lin,
           W_com, b_com, W_glob, b_glob):
    pad = NP - N
    x_p = jnp.pad(x, ((0, pad), (0, 0)))
    batch_p = jnp.pad(batch, (0, pad), constant_values=127).reshape(NP, 1)
    src = edge_index[0]
    dst = edge_index[1]
    row = lambda v: v.reshape(1, D)
    col = lambda v: v.reshape(NP, 1)

    a_s, a_d, amax, csum, ccnt, gsum = _tc1(
        x_p, batch_p, W_gat, row(att_src), row(att_dst))

    cnt0, cnt1 = _get_sc_hist()(dst)

    x2, c_row, com, gvec = _tc2(
        x_p, col(cnt0), col(cnt1), a_d, amax, csum, ccnt, gsum,
        W_com, row(b_com), W_glob, row(b_glob))

    aggx, gatw, x2a0, x2a1, den = _get_sc_main()(
        x_p, x2, src, dst, a_s.reshape(NP), a_d.reshape(NP),
        c_row.reshape(NP))

    (out_p,) = _tc3(
        x_p, aggx, gatw, x2a0, x2a1, col(den), col(cnt0), col(cnt1),
        col(a_s), col(a_d), col(c_row), batch_p, com, gvec,
        W_gcn, W_gat, W_sage_l, W_sage_r, W_gin, W_lin,
        row(b_gcn), row(b_gat), row(b_sage_l), row(b_gin), row(b_lin))

    return out_p[:N]


# speedup vs baseline: 36.0735x; 2.2004x over previous
import functools
import jax
import jax.numpy as jnp
from jax import lax
from jax.experimental import pallas as pl
from jax.experimental.pallas import tpu as pltpu
from jax.experimental.pallas import tpu_sc as plsc

N = 10000
NP = 10240
E = 320000
D = 128
COM = 100
BN = 512
GRID = NP // BN
RPT = NP // 16

K = 80
EPT1 = E // 16
CH1 = EPT1 // K
EPT2 = E // 32
CH2 = EPT2 // K
KH = 400
EH = E // 32
CHH = EH // KH

f32 = jnp.float32
i32 = jnp.int32


def _leaky(v):
    return jnp.where(v >= 0, v, 0.2 * v)


def _tc1_body(x_ref, b_ref, wgat_ref, asrc_ref, adst_ref,
              as_ref, ad_ref, amax_ref, csum_ref, ccnt_ref, gsum_ref):
    i = pl.program_id(0)
    xb = x_ref[...]
    xg = jnp.dot(xb, wgat_ref[...], preferred_element_type=f32)
    a_s = lax.dot_general(asrc_ref[...], xg, (((1,), (1,)), ((), ())),
                          preferred_element_type=f32)
    a_d = lax.dot_general(adst_ref[...], xg, (((1,), (1,)), ((), ())),
                          preferred_element_type=f32)
    as_ref[...] = a_s
    ad_ref[...] = a_d
    bcol = b_ref[...]
    oh = (bcol == lax.broadcasted_iota(i32, (BN, COM), 1)).astype(f32)
    csum_b = lax.dot_general(oh, xb, (((0,), (0,)), ((), ())),
                             preferred_element_type=f32)
    ccnt_b = lax.dot_general(oh, jnp.ones((BN, 1), f32),
                             (((0,), (0,)), ((), ())),
                             preferred_element_type=f32)
    gsum_b = jnp.sum(xb, axis=0, keepdims=True)
    a_b = jnp.maximum(jnp.max(a_s), 0.0).reshape(1, 1)

    @pl.when(i == 0)
    def _():
        csum_ref[...] = csum_b
        ccnt_ref[...] = ccnt_b
        gsum_ref[...] = gsum_b
        amax_ref[...] = a_b

    @pl.when(i > 0)
    def _():
        csum_ref[...] += csum_b
        ccnt_ref[...] += ccnt_b
        gsum_ref[...] += gsum_b
        amax_ref[...] = jnp.maximum(amax_ref[...], a_b)


_tc1 = pl.pallas_call(
    _tc1_body,
    grid=(GRID,),
    in_specs=[
        pl.BlockSpec((BN, D), lambda i: (i, 0)),
        pl.BlockSpec((BN, 1), lambda i: (i, 0)),
        pl.BlockSpec((D, D), lambda i: (0, 0)),
        pl.BlockSpec((1, D), lambda i: (0, 0)),
        pl.BlockSpec((1, D), lambda i: (0, 0)),
    ],
    out_specs=[
        pl.BlockSpec((1, BN), lambda i: (0, i)),
        pl.BlockSpec((1, BN), lambda i: (0, i)),
        pl.BlockSpec((1, 1), lambda i: (0, 0)),
        pl.BlockSpec((COM, D), lambda i: (0, 0)),
        pl.BlockSpec((COM, 1), lambda i: (0, 0)),
        pl.BlockSpec((1, D), lambda i: (0, 0)),
    ],
    out_shape=[
        jax.ShapeDtypeStruct((1, NP), f32),
        jax.ShapeDtypeStruct((1, NP), f32),
        jax.ShapeDtypeStruct((1, 1), f32),
        jax.ShapeDtypeStruct((COM, D), f32),
        jax.ShapeDtypeStruct((COM, 1), f32),
        jax.ShapeDtypeStruct((1, D), f32),
    ],
    compiler_params=pltpu.CompilerParams(
        dimension_semantics=("arbitrary",)),
)


def _tc2_body(x_ref, cnt0_ref, cnt1_ref, ad_ref, amax_ref,
              csum_ref, ccnt_ref, gsum_ref, wcom_ref, bcom_ref, wglob_ref,
              bglob_ref, x2_ref, c_ref, com_ref, gvec_ref):
    i = pl.program_id(0)
    cnt = cnt0_ref[...] + cnt1_ref[...]
    dinv = lax.rsqrt(cnt + 1.0)
    x2_ref[...] = x_ref[...] * dinv
    c_ref[...] = _leaky(amax_ref[0, 0] + ad_ref[...])

    @pl.when(i == 0)
    def _():
        cmean = csum_ref[...] / jnp.maximum(ccnt_ref[...], 1.0)
        com_ref[...] = jnp.dot(cmean, wcom_ref[...],
                               preferred_element_type=f32) + bcom_ref[...]
        gvec_ref[...] = jnp.dot(gsum_ref[...] * (1.0 / N), wglob_ref[...],
                                preferred_element_type=f32) + bglob_ref[...]


_tc2 = pl.pallas_call(
    _tc2_body,
    grid=(GRID,),
    in_specs=[
        pl.BlockSpec((BN, D), lambda i: (i, 0)),
        pl.BlockSpec((BN, 1), lambda i: (i, 0)),
        pl.BlockSpec((BN, 1), lambda i: (i, 0)),
        pl.BlockSpec((1, BN), lambda i: (0, i)),
        pl.BlockSpec((1, 1), lambda i: (0, 0)),
        pl.BlockSpec((COM, D), lambda i: (0, 0)),
        pl.BlockSpec((COM, 1), lambda i: (0, 0)),
        pl.BlockSpec((1, D), lambda i: (0, 0)),
        pl.BlockSpec((D, D), lambda i: (0, 0)),
        pl.BlockSpec((1, D), lambda i: (0, 0)),
        pl.BlockSpec((D, D), lambda i: (0, 0)),
        pl.BlockSpec((1, D), lambda i: (0, 0)),
    ],
    out_specs=[
        pl.BlockSpec((BN, D), lambda i: (i, 0)),
        pl.BlockSpec((1, BN), lambda i: (0, i)),
        pl.BlockSpec((COM, D), lambda i: (0, 0)),
        pl.BlockSpec((1, D), lambda i: (0, 0)),
    ],
    out_shape=[
        jax.ShapeDtypeStruct((NP, D), f32),
        jax.ShapeDtypeStruct((1, NP), f32),
        jax.ShapeDtypeStruct((COM, D), f32),
        jax.ShapeDtypeStruct((1, D), f32),
    ],
    compiler_params=pltpu.CompilerParams(
        dimension_semantics=("arbitrary",)),
)


def _sc_hist_body(dst_hbm, cnt0_hbm, cnt1_hbm, idx_v, ones_v, zb_v, acc_sh):
    cid = lax.axis_index("c")
    sid = lax.axis_index("s")
    wid = sid * 2 + cid
    for g in range(KH // 16):
        ones_v[pl.ds(g * 16, 16)] = jnp.ones((16,), f32)
    for g in range(RPT // 16):
        zb_v[pl.ds(g * 16, 16)] = jnp.zeros((16,), f32)
    pltpu.sync_copy(zb_v, acc_sh.at[pl.ds(sid * RPT, RPT)])
    plsc.subcore_barrier()

    def chunk(j, carry):
        base = wid * EH + j * KH
        pltpu.sync_copy(dst_hbm.at[pl.ds(base, KH)], idx_v)
        pltpu.sync_copy(ones_v, acc_sh.at[idx_v], add=True)
        return carry

    lax.fori_loop(0, CHH, chunk, 0)
    plsc.subcore_barrier()
    sl = pl.ds(sid * RPT, RPT)

    @pl.when(cid == 0)
    def _():
        pltpu.sync_copy(acc_sh.at[sl], cnt0_hbm.at[sl])

    @pl.when(cid == 1)
    def _():
        pltpu.sync_copy(acc_sh.at[sl], cnt1_hbm.at[sl])


@functools.lru_cache(maxsize=None)
def _get_sc_hist():
    return pl.kernel(
        _sc_hist_body,
        mesh=plsc.VectorSubcoreMesh(core_axis_name="c",
                                    subcore_axis_name="s"),
        out_type=[
            jax.ShapeDtypeStruct((NP,), f32),
            jax.ShapeDtypeStruct((NP,), f32),
        ],
        scratch_types=[
            pltpu.VMEM((KH,), i32),
            pltpu.VMEM((KH,), f32),
            pltpu.VMEM((RPT,), f32),
            pltpu.VMEM_SHARED((NP,), f32),
        ],
        compiler_params=pltpu.CompilerParams(needs_layout_passes=False,
                                             use_tc_tiling_on_sc=False),
    )


RB = K * D * 4
QB = K * 4
NBUF = 3


def _sc_main_body(x_hbm, x2_hbm, ed_hbm, as_hbm, ad_hbm, c_hbm,
                  aggx_hbm, gatw0_hbm, gatw1_hbm, x2a_hbm,
                  den0_hbm, den1_hbm,
                  edv0, edv1, edv2, srcv0, srcv1, srcv2,
                  dstv0, dstv1, dstv2, rows0, rows1, rows2,
                  wbuf0, wbuf1, wbuf2, asv0, asv1, asv2,
                  adv0, adv1, adv2, cv0, cv1, cv2,
                  gsem, qsem, ssem, dsem, acc_sh, den_sh):
    cid = lax.axis_index("c")
    sid = lax.axis_index("s")
    sl = pl.ds(sid * RPT, RPT)
    edv = [edv0, edv1, edv2]
    srcv = [srcv0, srcv1, srcv2]
    dstv = [dstv0, dstv1, dstv2]
    rows = [rows0, rows1, rows2]
    wbuf = [wbuf0, wbuf1, wbuf2]
    asv = [asv0, asv1, asv2]
    adv = [adv0, adv1, adv2]
    cv = [cv0, cv1, cv2]
    gsems = [gsem.at[b] for b in range(NBUF)]
    qsems = [qsem.at[b] for b in range(NBUF)]
    ssems = [ssem.at[b] for b in range(NBUF)]
    dsems = [dsem.at[b] for b in range(NBUF)]

    def zero_accs():
        def zrow(r, carry):
            for g in range(D // 16):
                rows0[r, pl.ds(g * 16, 16)] = jnp.zeros((16,), f32)
            return carry

        lax.fori_loop(0, K, zrow, 0)
        for g in range(K // 16):
            wbuf0[pl.ds(g * 16, 16)] = jnp.zeros((16,), f32)
        for j in range(RPT // K):
            pltpu.sync_copy(rows0, acc_sh.at[pl.ds(sid * RPT + j * K, K)])
            pltpu.sync_copy(wbuf0, den_sh.at[pl.ds(sid * RPT + j * K, K)])

    def pipe(chn, gbase, gather_hbm, weighted):
        def start_fetch(c, b):
            g = gbase(c)
            pltpu.sync_copy(ed_hbm.at[pl.ds(g * (2 * K), 2 * K)], edv[b])
            for j in range(K // 16):
                srcv[b][pl.ds(j * 16, 16)] = edv[b][pl.ds(j * 16, 16)]
                dstv[b][pl.ds(j * 16, 16)] = edv[b][pl.ds(K + j * 16, 16)]
            pltpu.async_copy(gather_hbm.at[srcv[b]], rows[b], gsems[b])
            if weighted:
                pltpu.async_copy(as_hbm.at[srcv[b]], asv[b], qsems[b])
                pltpu.async_copy(ad_hbm.at[dstv[b]], adv[b], qsems[b])
                pltpu.async_copy(c_hbm.at[dstv[b]], cv[b], qsems[b])

        def inner(b, c):
            pltpu.make_async_copy(gather_hbm.at[srcv[b]], rows[b],
                                  gsems[b]).wait()
            if weighted:
                pltpu.make_async_copy(as_hbm.at[srcv[b]], asv[b],
                                      qsems[b]).wait()
                pltpu.make_async_copy(ad_hbm.at[dstv[b]], adv[b],
                                      qsems[b]).wait()
                pltpu.make_async_copy(c_hbm.at[dstv[b]], cv[b],
                                      qsems[b]).wait()

                def grp(g, carry):
                    o = g * 16
                    ev = asv[b][pl.ds(o, 16)] + adv[b][pl.ds(o, 16)]
                    ev = jnp.where(ev >= 0, ev, 0.2 * ev)
                    w16 = jnp.exp(ev - cv[b][pl.ds(o, 16)])
                    wbuf[b][pl.ds(o, 16)] = w16
                    for j in range(16):
                        ws = jnp.full((16,), w16[j], f32)
                        for cg in range(D // 16):
                            rows[b][o + j, pl.ds(cg * 16, 16)] = \
                                rows[b][o + j, pl.ds(cg * 16, 16)] * ws
                    return carry

                lax.fori_loop(0, K // 16, grp, 0)
                pltpu.async_copy(wbuf[b], den_sh.at[dstv[b]], dsems[b],
                                 add=True)
            pltpu.async_copy(rows[b], acc_sh.at[dstv[b]], ssems[b], add=True)
            bp = (b + 2) % NBUF

            @pl.when(c >= 1)
            def _():
                pltpu.make_async_copy(rows[bp], acc_sh.at[dstv[bp]],
                                      ssems[bp]).wait()
                if weighted:
                    pltpu.make_async_copy(wbuf[bp], den_sh.at[dstv[bp]],
                                          dsems[bp]).wait()

            @pl.when(c + 2 < chn)
            def _():
                start_fetch(c + 2, bp)

        start_fetch(0, 0)
        start_fetch(1, 1)

        def outer(oi, carry):
            c0 = oi * NBUF
            for b in range(NBUF):
                c = c0 + b

                @pl.when(c < chn)
                def _():
                    inner(b, c)
            return carry

        lax.fori_loop(0, (chn + NBUF - 1) // NBUF, outer, 0)
        bl = (chn - 1) % NBUF
        pltpu.make_async_copy(rows[bl], acc_sh.at[dstv[bl]],
                              ssems[bl]).wait()
        if weighted:
            pltpu.make_async_copy(wbuf[bl], den_sh.at[dstv[bl]],
                                  dsems[bl]).wait()

    zero_accs()
    plsc.subcore_barrier()

    @pl.when(cid == 0)
    def _():
        pipe(CH1, lambda c: sid * CH1 + c, x_hbm, False)

    @pl.when(cid == 1)
    def _():
        pipe(CH2, lambda c: sid * CH2 + c, x_hbm, True)

    plsc.subcore_barrier()

    @pl.when(cid == 0)
    def _():
        pltpu.sync_copy(acc_sh.at[sl], aggx_hbm.at[sl])

    @pl.when(cid == 1)
    def _():
        pltpu.sync_copy(acc_sh.at[sl], gatw0_hbm.at[sl])
        pltpu.sync_copy(den_sh.at[sl], den0_hbm.at[sl])

    zero_accs()
    plsc.subcore_barrier()

    @pl.when(cid == 0)
    def _():
        pipe(CH2, lambda c: (E // (2 * K)) + sid * CH2 + c, x_hbm, True)

    @pl.when(cid == 1)
    def _():
        pipe(CH1, lambda c: sid * CH1 + c, x2_hbm, False)

    plsc.subcore_barrier()

    @pl.when(cid == 0)
    def _():
        pltpu.sync_copy(acc_sh.at[sl], gatw1_hbm.at[sl])
        pltpu.sync_copy(den_sh.at[sl], den1_hbm.at[sl])

    @pl.when(cid == 1)
    def _():
        pltpu.sync_copy(acc_sh.at[sl], x2a_hbm.at[sl])


@functools.lru_cache(maxsize=None)
def _get_sc_main():
    vi = pltpu.VMEM((2 * K,), i32)
    vk = pltpu.VMEM((K,), i32)
    vf = pltpu.VMEM((K,), f32)
    vr = pltpu.VMEM((K, D), f32)
    return pl.kernel(
        _sc_main_body,
        mesh=plsc.VectorSubcoreMesh(core_axis_name="c",
                                    subcore_axis_name="s"),
        out_type=[
            jax.ShapeDtypeStruct((NP, D), f32),
            jax.ShapeDtypeStruct((NP, D), f32),
            jax.ShapeDtypeStruct((NP, D), f32),
            jax.ShapeDtypeStruct((NP, D), f32),
            jax.ShapeDtypeStruct((NP,), f32),
            jax.ShapeDtypeStruct((NP,), f32),
        ],
        scratch_types=(
            [vi] * 3 + [vk] * 6 + [vr] * 3 + [vf] * 3
            + [vf] * 9
            + [pltpu.SemaphoreType.DMA((NBUF,))] * 4
            + [pltpu.VMEM_SHARED((NP, D), f32),
               pltpu.VMEM_SHARED((NP,), f32)]
        ),
        compiler_params=pltpu.CompilerParams(needs_layout_passes=False,
                                             use_tc_tiling_on_sc=False),
    )


def _tc3_body(x_ref, aggx_ref, gatw0_ref, gatw1_ref, x2a_ref,
              den0_ref, den1_ref,
              cnt0_ref, cnt1_ref, as_ref, ad_ref, c_ref, b_ref,
              com_ref, gvec_ref, wgcn_ref, wgat_ref,
              wsl_ref, wsr_ref, wgin_ref, wlin_ref,
              bgcn_ref, bgat_ref, bsl_ref, bgin_ref, blin_ref, out_ref):
    xb = x_ref[...]
    aggx = aggx_ref[...]
    cnt = cnt0_ref[...] + cnt1_ref[...]
    dinv = lax.rsqrt(cnt + 1.0)
    x2b = xb * dinv
    gcn = dinv * jnp.dot(x2a_ref[...] + x2b, wgcn_ref[...],
                         preferred_element_type=f32) + bgcn_ref[...]
    es = jnp.exp(_leaky(as_ref[...] + ad_ref[...]) - c_ref[...])
    den = den0_ref[...] + den1_ref[...]
    gat = jnp.dot(gatw0_ref[...] + gatw1_ref[...] + es * xb, wgat_ref[...],
                  preferred_element_type=f32) / (den + es) \
        + bgat_ref[...]
    invc = 1.0 / jnp.maximum(cnt, 1.0)
    sage = jnp.dot(aggx * invc, wsl_ref[...], preferred_element_type=f32) \
        + bsl_ref[...] \
        + jnp.dot(xb, wsr_ref[...], preferred_element_type=f32)
    gin = jnp.dot(xb + aggx, wgin_ref[...],
                  preferred_element_type=f32) + bgin_ref[...]
    lin = jnp.dot(xb, wlin_ref[...],
                  preferred_element_type=f32) + blin_ref[...]
    oh = (b_ref[...] == lax.broadcasted_iota(i32, (BN, COM), 1)).astype(f32)
    comm = jnp.dot(oh, com_ref[...], preferred_element_type=f32)
    glob = jnp.broadcast_to(gvec_ref[...], (BN, D))
    r = lambda v: jnp.maximum(v, 0.0)
    out_ref[...] = 0.12 * (r(gcn) + r(gat) + r(sage) + r(gin) + r(lin)) \
        + 0.3 * r(comm) + 0.1 * r(glob)


_col = pl.BlockSpec((BN, 1), lambda i: (i, 0))
_mat = pl.BlockSpec((BN, D), lambda i: (i, 0))
_wmat = pl.BlockSpec((D, D), lambda i: (0, 0))
_brow = pl.BlockSpec((1, D), lambda i: (0, 0))

_tc3 = pl.pallas_call(
    _tc3_body,
    grid=(GRID,),
    in_specs=[
        _mat, _mat, _mat, _mat, _mat,
        _col, _col, _col, _col, _col, _col, _col,
        pl.BlockSpec((BN, 1), lambda i: (i, 0)),
        pl.BlockSpec((COM, D), lambda i: (0, 0)),
        _brow,
        _wmat, _wmat, _wmat, _wmat, _wmat, _wmat,
        _brow, _brow, _brow, _brow, _brow,
    ],
    out_specs=[_mat],
    out_shape=[jax.ShapeDtypeStruct((NP, D), f32)],
    compiler_params=pltpu.CompilerParams(
        dimension_semantics=("arbitrary",)),
)


def kernel(x, edge_index, batch, W_gcn, b_gcn, W_gat, att_src, att_dst,
           b_gat, W_sage_l, b_sage_l, W_sage_r, W_gin, b_gin, W_lin, b_lin,
           W_com, b_com, W_glob, b_glob):
    pad = NP - N
    x_p = jnp.pad(x, ((0, pad), (0, 0)))
    batch_p = jnp.pad(batch, (0, pad), constant_values=127).reshape(NP, 1)
    src = edge_index[0]
    dst = edge_index[1]
    ed = jnp.concatenate(
        [src.reshape(E // K, K), dst.reshape(E // K, K)], axis=1
    ).reshape(2 * E)
    row = lambda v: v.reshape(1, D)
    col = lambda v: v.reshape(NP, 1)

    a_s, a_d, amax, csum, ccnt, gsum = _tc1(
        x_p, batch_p, W_gat, row(att_src), row(att_dst))

    cnt0, cnt1 = _get_sc_hist()(dst)

    x2, c_row, com, gvec = _tc2(
        x_p, col(cnt0), col(cnt1), a_d, amax, csum, ccnt, gsum,
        W_com, row(b_com), W_glob, row(b_glob))

    aggx, gatw0, gatw1, x2a, den0, den1 = _get_sc_main()(
        x_p, x2, ed, a_s.reshape(NP), a_d.reshape(NP),
        c_row.reshape(NP))

    (out_p,) = _tc3(
        x_p, aggx, gatw0, gatw1, x2a, col(den0), col(den1),
        col(cnt0), col(cnt1),
        col(a_s), col(a_d), col(c_row), batch_p, com, gvec,
        W_gcn, W_gat, W_sage_l, W_sage_r, W_gin, W_lin,
        row(b_gcn), row(b_gat), row(b_sage_l), row(b_gin), row(b_lin))

    return out_p[:N]


# speedup vs baseline: 39.2687x; 1.0886x over previous
import functools
import jax
import jax.numpy as jnp
from jax import lax
from jax.experimental import pallas as pl
from jax.experimental.pallas import tpu as pltpu
from jax.experimental.pallas import tpu_sc as plsc

N = 10000
NP = 10240
E = 320000
D = 128
COM = 100
BN = 512
GRID = NP // BN
RPT = NP // 16

K = 80
EPT1 = E // 16
CH1 = EPT1 // K
EPT2 = E // 32
CH2 = EPT2 // K
KH = 400
EH = E // 32
CHH = EH // KH

f32 = jnp.float32
i32 = jnp.int32


def _leaky(v):
    return jnp.where(v >= 0, v, 0.2 * v)


def _tc1_body(x_ref, b_ref, wgat_ref, asrc_ref, adst_ref,
              as_ref, ad_ref, amax_ref, csum_ref, ccnt_ref, gsum_ref):
    i = pl.program_id(0)
    xb = x_ref[...]
    xg = jnp.dot(xb, wgat_ref[...], preferred_element_type=f32)
    a_s = lax.dot_general(asrc_ref[...], xg, (((1,), (1,)), ((), ())),
                          preferred_element_type=f32)
    a_d = lax.dot_general(adst_ref[...], xg, (((1,), (1,)), ((), ())),
                          preferred_element_type=f32)
    as_ref[...] = a_s
    ad_ref[...] = a_d
    bcol = b_ref[...]
    oh = (bcol == lax.broadcasted_iota(i32, (BN, COM), 1)).astype(f32)
    csum_b = lax.dot_general(oh, xb, (((0,), (0,)), ((), ())),
                             preferred_element_type=f32)
    ccnt_b = lax.dot_general(oh, jnp.ones((BN, 1), f32),
                             (((0,), (0,)), ((), ())),
                             preferred_element_type=f32)
    gsum_b = jnp.sum(xb, axis=0, keepdims=True)
    a_b = jnp.maximum(jnp.max(a_s), 0.0).reshape(1, 1)

    @pl.when(i == 0)
    def _():
        csum_ref[...] = csum_b
        ccnt_ref[...] = ccnt_b
        gsum_ref[...] = gsum_b
        amax_ref[...] = a_b

    @pl.when(i > 0)
    def _():
        csum_ref[...] += csum_b
        ccnt_ref[...] += ccnt_b
        gsum_ref[...] += gsum_b
        amax_ref[...] = jnp.maximum(amax_ref[...], a_b)


_tc1 = pl.pallas_call(
    _tc1_body,
    grid=(GRID,),
    in_specs=[
        pl.BlockSpec((BN, D), lambda i: (i, 0)),
        pl.BlockSpec((BN, 1), lambda i: (i, 0)),
        pl.BlockSpec((D, D), lambda i: (0, 0)),
        pl.BlockSpec((1, D), lambda i: (0, 0)),
        pl.BlockSpec((1, D), lambda i: (0, 0)),
    ],
    out_specs=[
        pl.BlockSpec((1, BN), lambda i: (0, i)),
        pl.BlockSpec((1, BN), lambda i: (0, i)),
        pl.BlockSpec((1, 1), lambda i: (0, 0)),
        pl.BlockSpec((COM, D), lambda i: (0, 0)),
        pl.BlockSpec((COM, 1), lambda i: (0, 0)),
        pl.BlockSpec((1, D), lambda i: (0, 0)),
    ],
    out_shape=[
        jax.ShapeDtypeStruct((1, NP), f32),
        jax.ShapeDtypeStruct((1, NP), f32),
        jax.ShapeDtypeStruct((1, 1), f32),
        jax.ShapeDtypeStruct((COM, D), f32),
        jax.ShapeDtypeStruct((COM, 1), f32),
        jax.ShapeDtypeStruct((1, D), f32),
    ],
    compiler_params=pltpu.CompilerParams(
        dimension_semantics=("arbitrary",)),
)


def _tc2_body(x_ref, cnt0_ref, cnt1_ref, ad_ref, amax_ref,
              csum_ref, ccnt_ref, gsum_ref, wcom_ref, bcom_ref, wglob_ref,
              bglob_ref, x2_ref, c_ref, com_ref, gvec_ref):
    i = pl.program_id(0)
    cnt = cnt0_ref[...] + cnt1_ref[...]
    dinv = lax.rsqrt(cnt + 1.0)
    x2_ref[...] = x_ref[...] * dinv
    c_ref[...] = _leaky(amax_ref[0, 0] + ad_ref[...])

    @pl.when(i == 0)
    def _():
        cmean = csum_ref[...] / jnp.maximum(ccnt_ref[...], 1.0)
        com_ref[...] = jnp.dot(cmean, wcom_ref[...],
                               preferred_element_type=f32) + bcom_ref[...]
        gvec_ref[...] = jnp.dot(gsum_ref[...] * (1.0 / N), wglob_ref[...],
                                preferred_element_type=f32) + bglob_ref[...]


_tc2 = pl.pallas_call(
    _tc2_body,
    grid=(GRID,),
    in_specs=[
        pl.BlockSpec((BN, D), lambda i: (i, 0)),
        pl.BlockSpec((BN, 1), lambda i: (i, 0)),
        pl.BlockSpec((BN, 1), lambda i: (i, 0)),
        pl.BlockSpec((1, BN), lambda i: (0, i)),
        pl.BlockSpec((1, 1), lambda i: (0, 0)),
        pl.BlockSpec((COM, D), lambda i: (0, 0)),
        pl.BlockSpec((COM, 1), lambda i: (0, 0)),
        pl.BlockSpec((1, D), lambda i: (0, 0)),
        pl.BlockSpec((D, D), lambda i: (0, 0)),
        pl.BlockSpec((1, D), lambda i: (0, 0)),
        pl.BlockSpec((D, D), lambda i: (0, 0)),
        pl.BlockSpec((1, D), lambda i: (0, 0)),
    ],
    out_specs=[
        pl.BlockSpec((BN, D), lambda i: (i, 0)),
        pl.BlockSpec((1, BN), lambda i: (0, i)),
        pl.BlockSpec((COM, D), lambda i: (0, 0)),
        pl.BlockSpec((1, D), lambda i: (0, 0)),
    ],
    out_shape=[
        jax.ShapeDtypeStruct((NP, D), f32),
        jax.ShapeDtypeStruct((1, NP), f32),
        jax.ShapeDtypeStruct((COM, D), f32),
        jax.ShapeDtypeStruct((1, D), f32),
    ],
    compiler_params=pltpu.CompilerParams(
        dimension_semantics=("arbitrary",)),
)


def _sc_hist_body(dst_hbm, cnt0_hbm, cnt1_hbm, idx_v, ones_v, zb_v, acc_sh):
    cid = lax.axis_index("c")
    sid = lax.axis_index("s")
    wid = sid * 2 + cid
    for g in range(KH // 16):
        ones_v[pl.ds(g * 16, 16)] = jnp.ones((16,), f32)
    for g in range(RPT // 16):
        zb_v[pl.ds(g * 16, 16)] = jnp.zeros((16,), f32)
    pltpu.sync_copy(zb_v, acc_sh.at[pl.ds(sid * RPT, RPT)])
    plsc.subcore_barrier()

    def chunk(j, carry):
        base = wid * EH + j * KH
        pltpu.sync_copy(dst_hbm.at[pl.ds(base, KH)], idx_v)
        pltpu.sync_copy(ones_v, acc_sh.at[idx_v], add=True)
        return carry

    lax.fori_loop(0, CHH, chunk, 0)
    plsc.subcore_barrier()
    sl = pl.ds(sid * RPT, RPT)

    @pl.when(cid == 0)
    def _():
        pltpu.sync_copy(acc_sh.at[sl], cnt0_hbm.at[sl])

    @pl.when(cid == 1)
    def _():
        pltpu.sync_copy(acc_sh.at[sl], cnt1_hbm.at[sl])


@functools.lru_cache(maxsize=None)
def _get_sc_hist():
    return pl.kernel(
        _sc_hist_body,
        mesh=plsc.VectorSubcoreMesh(core_axis_name="c",
                                    subcore_axis_name="s"),
        out_type=[
            jax.ShapeDtypeStruct((NP,), f32),
            jax.ShapeDtypeStruct((NP,), f32),
        ],
        scratch_types=[
            pltpu.VMEM((KH,), i32),
            pltpu.VMEM((KH,), f32),
            pltpu.VMEM((RPT,), f32),
            pltpu.VMEM_SHARED((NP,), f32),
        ],
        compiler_params=pltpu.CompilerParams(needs_layout_passes=False,
                                             use_tc_tiling_on_sc=False),
    )


RB = K * D * 4
QB = K * 4
NBUF = 3
GB = 10


def _sc_main_body(x_hbm, x2_hbm, ed_hbm, as_hbm, ad_hbm, c_hbm,
                  aggx_hbm, gatw0_hbm, gatw1_hbm, x2a_hbm,
                  den0_hbm, den1_hbm,
                  edbuf, srcv0, srcv1, srcv2,
                  dstv0, dstv1, dstv2, rows0, rows1, rows2,
                  wbuf0, wbuf1, wbuf2, asv0, asv1, asv2,
                  adv0, adv1, adv2, cv0, cv1, cv2,
                  gsem, qsem, ssem, dsem, acc_sh, den_sh):
    cid = lax.axis_index("c")
    sid = lax.axis_index("s")
    sl = pl.ds(sid * RPT, RPT)
    srcv = [srcv0, srcv1, srcv2]
    dstv = [dstv0, dstv1, dstv2]
    rows = [rows0, rows1, rows2]
    wbuf = [wbuf0, wbuf1, wbuf2]
    asv = [asv0, asv1, asv2]
    adv = [adv0, adv1, adv2]
    cv = [cv0, cv1, cv2]
    gsems = [gsem.at[b] for b in range(NBUF)]
    qsems = [qsem.at[b] for b in range(NBUF)]
    ssems = [ssem.at[b] for b in range(NBUF)]
    dsems = [dsem.at[b] for b in range(NBUF)]

    def zero_accs():
        def zrow(r, carry):
            for g in range(D // 16):
                rows0[r, pl.ds(g * 16, 16)] = jnp.zeros((16,), f32)
            return carry

        lax.fori_loop(0, K, zrow, 0)
        for g in range(K // 16):
            wbuf0[pl.ds(g * 16, 16)] = jnp.zeros((16,), f32)
        for j in range(RPT // K):
            pltpu.sync_copy(rows0, acc_sh.at[pl.ds(sid * RPT + j * K, K)])
            pltpu.sync_copy(wbuf0, den_sh.at[pl.ds(sid * RPT + j * K, K)])

    def pipe(chn, gbase, gather_hbm, weighted):
        def start_fetch(c, b):
            cm = lax.rem(c, GB)

            @pl.when(cm == 0)
            def _():
                g = gbase(c)
                pltpu.sync_copy(ed_hbm.at[pl.ds(g * (2 * K), GB * 2 * K)],
                                edbuf)

            off = cm * (2 * K)
            for j in range(K // 16):
                srcv[b][pl.ds(j * 16, 16)] = edbuf[pl.ds(off + j * 16, 16)]
                dstv[b][pl.ds(j * 16, 16)] = \
                    edbuf[pl.ds(off + K + j * 16, 16)]
            pltpu.async_copy(gather_hbm.at[srcv[b]], rows[b], gsems[b])
            if weighted:
                pltpu.async_copy(as_hbm.at[srcv[b]], asv[b], qsems[b])
                pltpu.async_copy(ad_hbm.at[dstv[b]], adv[b], qsems[b])
                pltpu.async_copy(c_hbm.at[dstv[b]], cv[b], qsems[b])

        def inner(b, c):
            pltpu.make_async_copy(gather_hbm.at[srcv[b]], rows[b],
                                  gsems[b]).wait()
            if weighted:
                pltpu.make_async_copy(as_hbm.at[srcv[b]], asv[b],
                                      qsems[b]).wait()
                pltpu.make_async_copy(ad_hbm.at[dstv[b]], adv[b],
                                      qsems[b]).wait()
                pltpu.make_async_copy(c_hbm.at[dstv[b]], cv[b],
                                      qsems[b]).wait()

                def grp(g, carry):
                    o = g * 16
                    ev = asv[b][pl.ds(o, 16)] + adv[b][pl.ds(o, 16)]
                    ev = jnp.where(ev >= 0, ev, 0.2 * ev)
                    w16 = jnp.exp(ev - cv[b][pl.ds(o, 16)])
                    wbuf[b][pl.ds(o, 16)] = w16
                    for j in range(16):
                        ws = jnp.full((16,), w16[j], f32)
                        for cg in range(D // 16):
                            rows[b][o + j, pl.ds(cg * 16, 16)] = \
                                rows[b][o + j, pl.ds(cg * 16, 16)] * ws
                    return carry

                lax.fori_loop(0, K // 16, grp, 0)
                pltpu.async_copy(wbuf[b], den_sh.at[dstv[b]], dsems[b],
                                 add=True)
            pltpu.async_copy(rows[b], acc_sh.at[dstv[b]], ssems[b], add=True)
            bp = (b + 2) % NBUF

            @pl.when(c >= 1)
            def _():
                pltpu.make_async_copy(rows[bp], acc_sh.at[dstv[bp]],
                                      ssems[bp]).wait()
                if weighted:
                    pltpu.make_async_copy(wbuf[bp], den_sh.at[dstv[bp]],
                                          dsems[bp]).wait()

            @pl.when(c + 2 < chn)
            def _():
                start_fetch(c + 2, bp)

        start_fetch(0, 0)
        start_fetch(1, 1)

        def outer(oi, carry):
            c0 = oi * NBUF
            for b in range(NBUF):
                c = c0 + b

                @pl.when(c < chn)
                def _():
                    inner(b, c)
            return carry

        lax.fori_loop(0, (chn + NBUF - 1) // NBUF, outer, 0)
        bl = (chn - 1) % NBUF
        pltpu.make_async_copy(rows[bl], acc_sh.at[dstv[bl]],
                              ssems[bl]).wait()
        if weighted:
            pltpu.make_async_copy(wbuf[bl], den_sh.at[dstv[bl]],
                                  dsems[bl]).wait()

    zero_accs()
    plsc.subcore_barrier()

    @pl.when(cid == 0)
    def _():
        pipe(CH1, lambda c: sid * CH1 + c, x_hbm, False)

    @pl.when(cid == 1)
    def _():
        pipe(CH2, lambda c: sid * CH2 + c, x_hbm, True)

    plsc.subcore_barrier()

    @pl.when(cid == 0)
    def _():
        pltpu.sync_copy(acc_sh.at[sl], aggx_hbm.at[sl])

    @pl.when(cid == 1)
    def _():
        pltpu.sync_copy(acc_sh.at[sl], gatw0_hbm.at[sl])
        pltpu.sync_copy(den_sh.at[sl], den0_hbm.at[sl])

    zero_accs()
    plsc.subcore_barrier()

    @pl.when(cid == 0)
    def _():
        pipe(CH2, lambda c: (E // (2 * K)) + sid * CH2 + c, x_hbm, True)

    @pl.when(cid == 1)
    def _():
        pipe(CH1, lambda c: sid * CH1 + c, x2_hbm, False)

    plsc.subcore_barrier()

    @pl.when(cid == 0)
    def _():
        pltpu.sync_copy(acc_sh.at[sl], gatw1_hbm.at[sl])
        pltpu.sync_copy(den_sh.at[sl], den1_hbm.at[sl])

    @pl.when(cid == 1)
    def _():
        pltpu.sync_copy(acc_sh.at[sl], x2a_hbm.at[sl])


@functools.lru_cache(maxsize=None)
def _get_sc_main():
    vk = pltpu.VMEM((K,), i32)
    vf = pltpu.VMEM((K,), f32)
    vr = pltpu.VMEM((K, D), f32)
    return pl.kernel(
        _sc_main_body,
        mesh=plsc.VectorSubcoreMesh(core_axis_name="c",
                                    subcore_axis_name="s"),
        out_type=[
            jax.ShapeDtypeStruct((NP, D), f32),
            jax.ShapeDtypeStruct((NP, D), f32),
            jax.ShapeDtypeStruct((NP, D), f32),
            jax.ShapeDtypeStruct((NP, D), f32),
            jax.ShapeDtypeStruct((NP,), f32),
            jax.ShapeDtypeStruct((NP,), f32),
        ],
        scratch_types=(
            [pltpu.VMEM((GB * 2 * K,), i32)] + [vk] * 6 + [vr] * 3 + [vf] * 3
            + [vf] * 9
            + [pltpu.SemaphoreType.DMA((NBUF,))] * 4
            + [pltpu.VMEM_SHARED((NP, D), f32),
               pltpu.VMEM_SHARED((NP,), f32)]
        ),
        compiler_params=pltpu.CompilerParams(needs_layout_passes=False,
                                             use_tc_tiling_on_sc=False),
    )


def _tc3_body(x_ref, aggx_ref, gatw0_ref, gatw1_ref, x2a_ref,
              den0_ref, den1_ref,
              cnt0_ref, cnt1_ref, as_ref, ad_ref, c_ref, b_ref,
              com_ref, gvec_ref, wgcn_ref, wgat_ref,
              wsl_ref, wsr_ref, wgin_ref, wlin_ref,
              bgcn_ref, bgat_ref, bsl_ref, bgin_ref, blin_ref, out_ref):
    xb = x_ref[...]
    aggx = aggx_ref[...]
    cnt = cnt0_ref[...] + cnt1_ref[...]
    dinv = lax.rsqrt(cnt + 1.0)
    x2b = xb * dinv
    gcn = dinv * jnp.dot(x2a_ref[...] + x2b, wgcn_ref[...],
                         preferred_element_type=f32) + bgcn_ref[...]
    es = jnp.exp(_leaky(as_ref[...] + ad_ref[...]) - c_ref[...])
    den = den0_ref[...] + den1_ref[...]
    gat = jnp.dot(gatw0_ref[...] + gatw1_ref[...] + es * xb, wgat_ref[...],
                  preferred_element_type=f32) / (den + es) \
        + bgat_ref[...]
    invc = 1.0 / jnp.maximum(cnt, 1.0)
    sage = jnp.dot(aggx * invc, wsl_ref[...], preferred_element_type=f32) \
        + bsl_ref[...] \
        + jnp.dot(xb, wsr_ref[...], preferred_element_type=f32)
    gin = jnp.dot(xb + aggx, wgin_ref[...],
                  preferred_element_type=f32) + bgin_ref[...]
    lin = jnp.dot(xb, wlin_ref[...],
                  preferred_element_type=f32) + blin_ref[...]
    oh = (b_ref[...] == lax.broadcasted_iota(i32, (BN, COM), 1)).astype(f32)
    comm = jnp.dot(oh, com_ref[...], preferred_element_type=f32)
    glob = jnp.broadcast_to(gvec_ref[...], (BN, D))
    r = lambda v: jnp.maximum(v, 0.0)
    out_ref[...] = 0.12 * (r(gcn) + r(gat) + r(sage) + r(gin) + r(lin)) \
        + 0.3 * r(comm) + 0.1 * r(glob)


_col = pl.BlockSpec((BN, 1), lambda i: (i, 0))
_mat = pl.BlockSpec((BN, D), lambda i: (i, 0))
_wmat = pl.BlockSpec((D, D), lambda i: (0, 0))
_brow = pl.BlockSpec((1, D), lambda i: (0, 0))

_tc3 = pl.pallas_call(
    _tc3_body,
    grid=(GRID,),
    in_specs=[
        _mat, _mat, _mat, _mat, _mat,
        _col, _col, _col, _col, _col, _col, _col,
        pl.BlockSpec((BN, 1), lambda i: (i, 0)),
        pl.BlockSpec((COM, D), lambda i: (0, 0)),
        _brow,
        _wmat, _wmat, _wmat, _wmat, _wmat, _wmat,
        _brow, _brow, _brow, _brow, _brow,
    ],
    out_specs=[_mat],
    out_shape=[jax.ShapeDtypeStruct((NP, D), f32)],
    compiler_params=pltpu.CompilerParams(
        dimension_semantics=("arbitrary",)),
)


def kernel(x, edge_index, batch, W_gcn, b_gcn, W_gat, att_src, att_dst,
           b_gat, W_sage_l, b_sage_l, W_sage_r, W_gin, b_gin, W_lin, b_lin,
           W_com, b_com, W_glob, b_glob):
    pad = NP - N
    x_p = jnp.pad(x, ((0, pad), (0, 0)))
    batch_p = jnp.pad(batch, (0, pad), constant_values=127).reshape(NP, 1)
    src = edge_index[0]
    dst = edge_index[1]
    ed = jnp.concatenate(
        [src.reshape(E // K, K), dst.reshape(E // K, K)], axis=1
    ).reshape(2 * E)
    ed = jnp.pad(ed, (0, GB * 2 * K))
    row = lambda v: v.reshape(1, D)
    col = lambda v: v.reshape(NP, 1)

    a_s, a_d, amax, csum, ccnt, gsum = _tc1(
        x_p, batch_p, W_gat, row(att_src), row(att_dst))

    cnt0, cnt1 = _get_sc_hist()(dst)

    x2, c_row, com, gvec = _tc2(
        x_p, col(cnt0), col(cnt1), a_d, amax, csum, ccnt, gsum,
        W_com, row(b_com), W_glob, row(b_glob))

    aggx, gatw0, gatw1, x2a, den0, den1 = _get_sc_main()(
        x_p, x2, ed, a_s.reshape(NP), a_d.reshape(NP),
        c_row.reshape(NP))

    (out_p,) = _tc3(
        x_p, aggx, gatw0, gatw1, x2a, col(den0), col(den1),
        col(cnt0), col(cnt1),
        col(a_s), col(a_d), col(c_row), batch_p, com, gvec,
        W_gcn, W_gat, W_sage_l, W_sage_r, W_gin, W_lin,
        row(b_gcn), row(b_gat), row(b_sage_l), row(b_gin), row(b_lin))

    return out_p[:N]
